# Initial kernel scaffold; baseline (speedup 1.0000x reference)
import jax
import jax.numpy as jnp
from jax.experimental import pallas as pl


def kernel(query_size, key_size, x_emb, y_emb):
    raise NotImplementedError("write your pallas kernel here")


# speedup vs baseline: 18.1040x; 18.1040x over previous
import jax
import jax.numpy as jnp
from jax.experimental import pallas as pl


def _outer_sum_kernel(x_ref, y_ref, o_ref):
    o_ref[...] = x_ref[...][:, None, :] + y_ref[...][None, :, :]


def kernel(query_size, key_size, x_emb, y_emb):
    q, d = x_emb.shape
    k, _ = y_emb.shape
    x_eff = jnp.take(x_emb, jnp.arange(q) + (query_size - q), axis=0)
    y_eff = jnp.take(y_emb, jnp.arange(k) + (key_size - k), axis=0)
    bq = 16
    return pl.pallas_call(
        _outer_sum_kernel,
        grid=(q // bq,),
        in_specs=[
            pl.BlockSpec((bq, d), lambda i: (i, 0)),
            pl.BlockSpec((k, d), lambda i: (0, 0)),
        ],
        out_specs=pl.BlockSpec((bq, k, d), lambda i: (i, 0, 0)),
        out_shape=jax.ShapeDtypeStruct((q, k, d), x_emb.dtype),
    )(x_eff, y_eff)


# speedup vs baseline: 59.1318x; 3.2662x over previous
import jax
import jax.numpy as jnp
from jax.experimental import pallas as pl


def _outer_sum_kernel(x_ref, y_ref, o_ref):
    o_ref[...] = x_ref[...][:, None, :] + y_ref[...][None, :, :]


def kernel(query_size, key_size, x_emb, y_emb):
    q, d = x_emb.shape
    k, _ = y_emb.shape
    x_eff = jnp.take(x_emb, jnp.arange(q) + (query_size - q), axis=0)
    y_eff = jnp.take(y_emb, jnp.arange(k) + (key_size - k), axis=0)

    pack = 128 // d
    kp = k // pack
    x_t = jnp.tile(x_eff, (1, pack))
    y_t = y_eff.reshape(kp, pack * d)

    bq = 64
    out3 = pl.pallas_call(
        _outer_sum_kernel,
        grid=(q // bq,),
        in_specs=[
            pl.BlockSpec((bq, pack * d), lambda i: (i, 0)),
            pl.BlockSpec((kp, pack * d), lambda i: (0, 0)),
        ],
        out_specs=pl.BlockSpec((bq, kp, pack * d), lambda i: (i, 0, 0)),
        out_shape=jax.ShapeDtypeStruct((q, kp, pack * d), x_emb.dtype),
    )(x_t, y_t)
    return out3.reshape(q, k, d)
